# Initial kernel scaffold; baseline (speedup 1.0000x reference)
import jax
import jax.numpy as jnp
from jax.experimental import pallas as pl


def kernel(xb, gt, grid, x_idx, y_buf):
    raise NotImplementedError("write your pallas kernel here")


# speedup vs baseline: 28.9864x; 28.9864x over previous
import functools

import jax
import jax.numpy as jnp
from jax import lax
from jax.experimental import pallas as pl
from jax.experimental.pallas import tpu as pltpu
from jax.experimental.pallas import tpu_sc as plsc

NPATCH = 18
P = 128
IMG_H = 512
IMG_W = 1024
B = 8
C = 3
R = NPATCH * P
HW = IMG_H * IMG_W
NC = 2
NS = 16
NW = NC * NS
JW = 32

_MESH = plsc.VectorSubcoreMesh(core_axis_name="c", subcore_axis_name="s")

_i32 = jnp.int32
_f32 = jnp.float32


def _wid():
    return lax.axis_index("s") * NC + lax.axis_index("c")


def _iota16():
    return lax.iota(_i32, 16)


ROWS_PER_W = R // NW
ROW_TILE = 8
N_TILES = ROWS_PER_W // ROW_TILE
ZCHUNK = 16384
ACC_PER_TILE = HW // NS


def _pre_body(ybuf_hbm, xidx_hbm, w_hbm, tgt_hbm, cnt2_hbm,
              ybrows, xirows, wrows, trows, cntbuf, idx2, ones128, zbuf,
              cnt2acc):
    wid = _wid()
    sid = lax.axis_index("s")
    core = lax.axis_index("c")
    ones = jnp.ones((16,), _f32)
    zf = jnp.zeros((16,), _f32)

    for q in range(8):
        ones128[pl.ds(q * 16, 16)] = ones

    def _zb(q, carry):
        zbuf[pl.ds(q * 16, 16)] = zf
        return carry
    lax.fori_loop(0, ZCHUNK // 16, _zb, 0)

    for q in range(ACC_PER_TILE // ZCHUNK):
        pltpu.sync_copy(zbuf, cnt2acc.at[pl.ds(sid * ACC_PER_TILE + q * ZCHUNK, ZCHUNK)])
    plsc.subcore_barrier()

    def _tile(t, carry):
        r0 = wid * ROWS_PER_W + t * ROW_TILE
        pltpu.sync_copy(ybuf_hbm.at[pl.ds(r0, ROW_TILE)], ybrows)
        pltpu.sync_copy(xidx_hbm.at[pl.ds(r0, ROW_TILE)], xirows)
        for rr in range(ROW_TILE):
            for q in range(IMG_W // 16):
                cntbuf[pl.ds(q * 16, 16)] = zf
            for q in range(P // 16):
                xi = xirows[rr, pl.ds(q * 16, 16)]
                plsc.addupdate_scatter(cntbuf, [xi], ones)
            rsel = jnp.full((16,), rr, _i32)
            for q in range(P // 16):
                xi = xirows[rr, pl.ds(q * 16, 16)]
                cnt = plsc.load_gather(cntbuf, [xi])
                wrows[rr, pl.ds(q * 16, 16)] = 1.0 / cnt
                yb = plsc.load_gather(ybrows, [rsel, xi])
                trows[rr, pl.ds(q * 16, 16)] = yb * IMG_W + xi
            for s in range(8):
                for q2 in range(8):
                    off = s * 128 + q2 * 16
                    yb = ybrows[rr, pl.ds(off, 16)]
                    idx2[s, pl.ds(q2 * 16, 16)] = yb * IMG_W + (off + _iota16())
            for s in range(8):
                pltpu.sync_copy(ones128, cnt2acc.at[idx2.at[s]], add=True)
        pltpu.sync_copy(wrows, w_hbm.at[pl.ds(r0, ROW_TILE)])
        pltpu.sync_copy(trows, tgt_hbm.at[pl.ds(r0, ROW_TILE)])
        return carry

    lax.fori_loop(0, N_TILES, _tile, 0)
    plsc.subcore_barrier()
    for q in range(ACC_PER_TILE // ZCHUNK):
        off = sid * ACC_PER_TILE + q * ZCHUNK
        pltpu.sync_copy(cnt2acc.at[pl.ds(off, ZCHUNK)],
                        cnt2_hbm.at[core, pl.ds(off, ZCHUNK)])


_sc_pre = pl.kernel(
    _pre_body,
    out_type=(
        jax.ShapeDtypeStruct((R, P), _f32),
        jax.ShapeDtypeStruct((R, P), _i32),
        jax.ShapeDtypeStruct((NC, HW), _f32),
    ),
    mesh=_MESH,
    compiler_params=pltpu.CompilerParams(needs_layout_passes=False),
    scratch_types=[
        pltpu.VMEM((ROW_TILE, IMG_W), _i32),
        pltpu.VMEM((ROW_TILE, P), _i32),
        pltpu.VMEM((ROW_TILE, P), _f32),
        pltpu.VMEM((ROW_TILE, P), _i32),
        pltpu.VMEM((IMG_W,), _f32),
        pltpu.VMEM((8, 128), _i32),
        pltpu.VMEM((128,), _f32),
        pltpu.VMEM((ZCHUNK,), _f32),
        pltpu.VMEM_SHARED((HW,), _f32),
    ],
)


def _stage_quarter(imgT_hbm, m, k, h, xA_hbm, xB_hbm, wxA_hbm, wxB_hbm,
                   y0F_hbm, fyF_hbm, idxa, idxb, wxa_v, wxb_v, y0v, fyv,
                   stg, sem):
    pltpu.sync_copy(xA_hbm.at[k, h], idxa)
    pltpu.sync_copy(xB_hbm.at[k, h], idxb)
    pltpu.sync_copy(wxA_hbm.at[k, h], wxa_v)
    pltpu.sync_copy(wxB_hbm.at[k, h], wxb_v)
    pltpu.sync_copy(y0F_hbm.at[k, h], y0v)
    pltpu.sync_copy(fyF_hbm.at[k, h], fyv)
    pltpu.async_copy(imgT_hbm.at[m].at[idxa], stg.at[0], sem).wait()
    pltpu.async_copy(imgT_hbm.at[m].at[idxb], stg.at[1], sem).wait()


def _blend_group(g, q, stg, y0v, fyv, wxa_v, wxb_v):
    col0 = q * 16
    jl0 = (q % 2) * 16
    y0 = y0v[g, pl.ds(col0, 16)]
    fy = fyv[g, pl.ds(col0, 16)]
    y1 = jnp.minimum(y0 + 1, IMG_H - 1)
    wy1 = jnp.where(y0 <= IMG_H - 2, fy, 0.0)
    wy0 = 1.0 - fy
    jlv = jl0 + _iota16()
    z16 = jnp.zeros((16,), _i32)
    o16 = jnp.ones((16,), _i32)
    a00 = plsc.load_gather(stg, [z16, jlv, y0])
    a01 = plsc.load_gather(stg, [z16, jlv, y1])
    b00 = plsc.load_gather(stg, [o16, jlv, y0])
    b01 = plsc.load_gather(stg, [o16, jlv, y1])
    wxa = wxa_v[pl.ds(jl0, 16)]
    wxb = wxb_v[pl.ds(jl0, 16)]
    return (a00 * wy0 + a01 * wy1) * wxa + (b00 * wy0 + b01 * wy1) * wxb


XB_UNITS = B * C * NPATCH
XB_T = (XB_UNITS + NW - 1) // NW


def _xb_body(imgT_hbm, xA_hbm, xB_hbm, wxA_hbm, wxB_hbm, y0F_hbm, fyF_hbm,
             out_hbm, idxa, idxb, wxa_v, wxb_v, y0v, fyv, stg, outF, sem):
    wid = _wid()

    def _unit(t, carry):
        u = wid + NW * t

        @pl.when(u < XB_UNITS)
        def _():
            m = u // NPATCH
            k = u % NPATCH
            for h in range(4):
                _stage_quarter(imgT_hbm, m, k, h, xA_hbm, xB_hbm, wxA_hbm,
                               wxB_hbm, y0F_hbm, fyF_hbm, idxa, idxb, wxa_v,
                               wxb_v, y0v, fyv, stg, sem)

                def _grp(g, carry2):
                    for q in range(8):
                        res = _blend_group(g, q, stg, y0v, fyv, wxa_v, wxb_v)
                        i_row = 4 * g + (q // 2)
                        outF[i_row, pl.ds((q % 2) * 16, 16)] = res
                    return carry2
                lax.fori_loop(0, 32, _grp, 0)
                pltpu.sync_copy(outF, out_hbm.at[m, k, h])
        return carry

    lax.fori_loop(0, XB_T, _unit, 0)


_sc_xb = pl.kernel(
    _xb_body,
    out_type=jax.ShapeDtypeStruct((B * C, NPATCH, 4, P, JW), _f32),
    mesh=_MESH,
    compiler_params=pltpu.CompilerParams(needs_layout_passes=False),
    scratch_types=[
        pltpu.VMEM((JW,), _i32),
        pltpu.VMEM((JW,), _i32),
        pltpu.VMEM((JW,), _f32),
        pltpu.VMEM((JW,), _f32),
        pltpu.VMEM((JW, 128), _i32),
        pltpu.VMEM((JW, 128), _f32),
        pltpu.VMEM((2, JW, IMG_H), _f32),
        pltpu.VMEM((P, JW), _f32),
        pltpu.SemaphoreType.DMA,
    ],
)


GT_UNITS = NPATCH * 4
GT_T = (GT_UNITS + NW - 1) // NW
ZCHUNK_GT = 8192


def _gt_body(gtT_hbm, xA_hbm, xB_hbm, wxA_hbm, wxB_hbm, y0F_hbm, fyF_hbm,
             wF_hbm, tgtF_hbm, accp_hbm, idxa, idxb, wxa_v, wxb_v, y0v, fyv,
             stg, wFv, tgtv, valsS, zbuf, acc, sem):
    wid = _wid()
    sid = lax.axis_index("s")
    core = lax.axis_index("c")
    zf = jnp.zeros((16,), _f32)

    def _zb(q, carry):
        zbuf[pl.ds(q * 16, 16)] = zf
        return carry
    lax.fori_loop(0, ZCHUNK_GT // 16, _zb, 0)

    def _batch(b, carry):
        for q in range(ACC_PER_TILE // ZCHUNK_GT):
            pltpu.sync_copy(zbuf, acc.at[pl.ds(sid * ACC_PER_TILE + q * ZCHUNK_GT, ZCHUNK_GT)])
        plsc.subcore_barrier()
        for t in range(GT_T):
            su = wid + NW * t

            @pl.when(su < GT_UNITS)
            def _():
                k = su // 4
                h = su % 4
                _stage_quarter(gtT_hbm, b, k, h, xA_hbm, xB_hbm, wxA_hbm,
                               wxB_hbm, y0F_hbm, fyF_hbm, idxa, idxb, wxa_v,
                               wxb_v, y0v, fyv, stg, sem)
                pltpu.sync_copy(wF_hbm.at[k, h], wFv)
                pltpu.sync_copy(tgtF_hbm.at[k, h], tgtv)

                def _grp(g, carry2):
                    for q in range(8):
                        res = _blend_group(g, q, stg, y0v, fyv, wxa_v, wxb_v)
                        res = res * wFv[g, pl.ds(q * 16, 16)]
                        valsS[g, pl.ds(q * 16, 16)] = res
                    return carry2
                lax.fori_loop(0, 32, _grp, 0)

                def _scat(g, carry2):
                    pltpu.sync_copy(valsS.at[g], acc.at[tgtv.at[g]], add=True)
                    return carry2
                lax.fori_loop(0, 32, _scat, 0)
        plsc.subcore_barrier()
        for q in range(ACC_PER_TILE // ZCHUNK_GT):
            off = sid * ACC_PER_TILE + q * ZCHUNK_GT
            pltpu.sync_copy(acc.at[pl.ds(off, ZCHUNK_GT)],
                            accp_hbm.at[core, b, pl.ds(off, ZCHUNK_GT)])
        plsc.subcore_barrier()
        return carry

    lax.fori_loop(0, B, _batch, 0)


_sc_gt = pl.kernel(
    _gt_body,
    out_type=jax.ShapeDtypeStruct((NC, B, HW), _f32),
    mesh=_MESH,
    compiler_params=pltpu.CompilerParams(needs_layout_passes=False),
    scratch_types=[
        pltpu.VMEM((JW,), _i32),
        pltpu.VMEM((JW,), _i32),
        pltpu.VMEM((JW,), _f32),
        pltpu.VMEM((JW,), _f32),
        pltpu.VMEM((JW, 128), _i32),
        pltpu.VMEM((JW, 128), _f32),
        pltpu.VMEM((2, JW, IMG_H), _f32),
        pltpu.VMEM((JW, 128), _f32),
        pltpu.VMEM((JW, 128), _i32),
        pltpu.VMEM((JW, 128), _f32),
        pltpu.VMEM((ZCHUNK_GT,), _f32),
        pltpu.VMEM_SHARED((HW,), _f32),
        pltpu.SemaphoreType.DMA,
    ],
)


ROWCHUNK = 128


def _fin_body(accp_ref, cnt2_ref, out_ref):
    num = accp_ref[0, 0] + accp_ref[1, 0]
    den = jnp.maximum(cnt2_ref[0] + cnt2_ref[1], 1.0)
    out_ref[...] = (num / den)[None]


def _tc_fin(accp, cnt2p):
    nchunk = IMG_H // ROWCHUNK
    accp = accp.reshape(NC, B, IMG_H, IMG_W)
    cnt2p = cnt2p.reshape(NC, IMG_H, IMG_W)
    return pl.pallas_call(
        _fin_body,
        grid=(B, nchunk),
        in_specs=[
            pl.BlockSpec((NC, 1, ROWCHUNK, IMG_W), lambda b, j: (0, b, j, 0)),
            pl.BlockSpec((NC, ROWCHUNK, IMG_W), lambda b, j: (0, j, 0)),
        ],
        out_specs=pl.BlockSpec((1, ROWCHUNK, IMG_W), lambda b, j: (b, j, 0)),
        out_shape=jax.ShapeDtypeStruct((B, IMG_H, IMG_W), _f32),
    )(accp, cnt2p)


def _flatten_q(a):
    return (a.reshape(NPATCH, P, 4, JW).transpose(0, 2, 1, 3)
             .reshape(NPATCH, 4, JW, 128))


def kernel(xb, gt, grid, x_idx, y_buf):
    g3 = grid.reshape(NPATCH, P, P, 2)
    gxk = g3[:, 0, :, 0]
    gyk = g3[..., 1]

    px = (gxk + 1.0) * 0.5 * (IMG_W - 1)
    x0 = jnp.floor(px)
    fx = px - x0
    x0i = x0.astype(_i32)
    x1i = x0i + 1
    wx0 = 1.0 - fx
    wx1 = jnp.where(x1i <= IMG_W - 1, fx, 0.0)
    x1c = jnp.clip(x1i, 0, IMG_W - 1)

    py = (gyk + 1.0) * 0.5 * (IMG_H - 1)
    y0 = jnp.floor(py)
    fy = (py - y0).astype(_f32)
    y0i = y0.astype(_i32)

    xA = x0i.reshape(NPATCH, 4, JW)
    xB = x1c.reshape(NPATCH, 4, JW)
    wxA = wx0.reshape(NPATCH, 4, JW).astype(_f32)
    wxB = wx1.reshape(NPATCH, 4, JW).astype(_f32)
    y0F = _flatten_q(y0i)
    fyF = _flatten_q(fy)

    xbT = xb.reshape(B * C, IMG_H, IMG_W).transpose(0, 2, 1)
    gtT = gt.transpose(0, 2, 1)

    w2304, tgt2304, cnt2p = _sc_pre(y_buf, x_idx)
    wF = _flatten_q(w2304.reshape(NPATCH, P, P))
    tgtF = _flatten_q(tgt2304.reshape(NPATCH, P, P))

    out1h = _sc_xb(xbT, xA, xB, wxA, wxB, y0F, fyF)
    out1 = (out1h.reshape(B, C, NPATCH, 4, P, JW)
                 .transpose(0, 1, 2, 4, 3, 5)
                 .reshape(B, C, NPATCH, P, P))

    accp = _sc_gt(gtT, xA, xB, wxA, wxB, y0F, fyF, wF, tgtF)
    out2 = _tc_fin(accp, cnt2p)
    return out1, out2


# speedup vs baseline: 37.0263x; 1.2774x over previous
import functools

import jax
import jax.numpy as jnp
from jax import lax
from jax.experimental import pallas as pl
from jax.experimental.pallas import tpu as pltpu
from jax.experimental.pallas import tpu_sc as plsc

NPATCH = 18
P = 128
IMG_H = 512
IMG_W = 1024
B = 8
C = 3
R = NPATCH * P
HW = IMG_H * IMG_W
NC = 2
NS = 16
NW = NC * NS
JW = 32

_MESH = plsc.VectorSubcoreMesh(core_axis_name="c", subcore_axis_name="s")

_i32 = jnp.int32
_f32 = jnp.float32


def _wid():
    return lax.axis_index("s") * NC + lax.axis_index("c")


def _iota16():
    return lax.iota(_i32, 16)


ROWS_PER_W = R // NW
ROW_TILE = 8
N_TILES = ROWS_PER_W // ROW_TILE
ZCHUNK = 16384
ACC_PER_TILE = HW // NS


def _pre_body(ybuf_hbm, xidx_hbm, w_hbm, tgt_hbm, cnt2_hbm,
              ybrows, xirows, wrows, trows, cntbuf, idx2, ones128, zbuf,
              cnt2acc):
    wid = _wid()
    sid = lax.axis_index("s")
    core = lax.axis_index("c")
    ones = jnp.ones((16,), _f32)
    zf = jnp.zeros((16,), _f32)

    for q in range(8):
        ones128[pl.ds(q * 16, 16)] = ones

    def _zb(q, carry):
        zbuf[pl.ds(q * 16, 16)] = zf
        return carry
    lax.fori_loop(0, ZCHUNK // 16, _zb, 0)

    for q in range(ACC_PER_TILE // ZCHUNK):
        pltpu.sync_copy(zbuf, cnt2acc.at[pl.ds(sid * ACC_PER_TILE + q * ZCHUNK, ZCHUNK)])
    plsc.subcore_barrier()

    def _tile(t, carry):
        r0 = wid * ROWS_PER_W + t * ROW_TILE
        pltpu.sync_copy(ybuf_hbm.at[pl.ds(r0, ROW_TILE)], ybrows)
        pltpu.sync_copy(xidx_hbm.at[pl.ds(r0, ROW_TILE)], xirows)
        for rr in range(ROW_TILE):
            for q in range(IMG_W // 16):
                cntbuf[pl.ds(q * 16, 16)] = zf
            for q in range(P // 16):
                xi = xirows[rr, pl.ds(q * 16, 16)]
                plsc.addupdate_scatter(cntbuf, [xi], ones)
            rsel = jnp.full((16,), rr, _i32)
            for q in range(P // 16):
                xi = xirows[rr, pl.ds(q * 16, 16)]
                cnt = plsc.load_gather(cntbuf, [xi])
                wrows[rr, pl.ds(q * 16, 16)] = 1.0 / cnt
                yb = plsc.load_gather(ybrows, [rsel, xi])
                trows[rr, pl.ds(q * 16, 16)] = yb * IMG_W + xi
            for s in range(8):
                for q2 in range(8):
                    off = s * 128 + q2 * 16
                    yb = ybrows[rr, pl.ds(off, 16)]
                    idx2[s, pl.ds(q2 * 16, 16)] = yb * IMG_W + (off + _iota16())
            for s in range(8):
                pltpu.sync_copy(ones128, cnt2acc.at[idx2.at[s]], add=True)
        pltpu.sync_copy(wrows, w_hbm.at[pl.ds(r0, ROW_TILE)])
        pltpu.sync_copy(trows, tgt_hbm.at[pl.ds(r0, ROW_TILE)])
        return carry

    lax.fori_loop(0, N_TILES, _tile, 0)
    plsc.subcore_barrier()
    for q in range(ACC_PER_TILE // ZCHUNK):
        off = sid * ACC_PER_TILE + q * ZCHUNK
        pltpu.sync_copy(cnt2acc.at[pl.ds(off, ZCHUNK)],
                        cnt2_hbm.at[core, pl.ds(off, ZCHUNK)])


_sc_pre = pl.kernel(
    _pre_body,
    out_type=(
        jax.ShapeDtypeStruct((R, P), _f32),
        jax.ShapeDtypeStruct((R, P), _i32),
        jax.ShapeDtypeStruct((NC, HW), _f32),
    ),
    mesh=_MESH,
    compiler_params=pltpu.CompilerParams(needs_layout_passes=False),
    scratch_types=[
        pltpu.VMEM((ROW_TILE, IMG_W), _i32),
        pltpu.VMEM((ROW_TILE, P), _i32),
        pltpu.VMEM((ROW_TILE, P), _f32),
        pltpu.VMEM((ROW_TILE, P), _i32),
        pltpu.VMEM((IMG_W,), _f32),
        pltpu.VMEM((8, 128), _i32),
        pltpu.VMEM((128,), _f32),
        pltpu.VMEM((ZCHUNK,), _f32),
        pltpu.VMEM_SHARED((HW,), _f32),
    ],
)


def _blend_group(g, q, stg, yv, xv, yoff):
    col0 = q * 16
    jl0 = (q % 2) * 16
    y0 = yv[yoff + g, pl.ds(col0, 16)]
    fy = plsc.bitcast(yv[yoff + 32 + g, pl.ds(col0, 16)], _f32)
    y1 = jnp.minimum(y0 + 1, IMG_H - 1)
    wy1 = jnp.where(y0 <= IMG_H - 2, fy, 0.0)
    wy0 = 1.0 - fy
    jlv = jl0 + _iota16()
    z16 = jnp.zeros((16,), _i32)
    o16 = jnp.ones((16,), _i32)
    a00 = plsc.load_gather(stg, [z16, jlv, y0])
    a01 = plsc.load_gather(stg, [z16, jlv, y1])
    b00 = plsc.load_gather(stg, [o16, jlv, y0])
    b01 = plsc.load_gather(stg, [o16, jlv, y1])
    wxa = plsc.bitcast(xv[2, pl.ds(jl0, 16)], _f32)
    wxb = plsc.bitcast(xv[3, pl.ds(jl0, 16)], _f32)
    return (a00 * wy0 + a01 * wy1) * wxa + (b00 * wy0 + b01 * wy1) * wxb


XB_Q = B * C * NPATCH * 4
XB_QPW = XB_Q // NW


def _xb_body(imgT_hbm, xpk_hbm, ypk_hbm, out_hbm,
             xv, yv, stg, outF, sem):
    wid = _wid()

    def _q(s, carry):
        qid = wid * XB_QPW + s
        kh = qid // (B * C)
        m = qid % (B * C)
        k = kh // 4
        h = kh % 4

        @pl.when(jnp.logical_or(s == 0, m == 0))
        def _():
            pltpu.sync_copy(xpk_hbm.at[k, h], xv)
            pltpu.sync_copy(ypk_hbm.at[k, h], yv)
        da = pltpu.async_copy(imgT_hbm.at[m].at[xv.at[0]], stg.at[0], sem)
        db = pltpu.async_copy(imgT_hbm.at[m].at[xv.at[1]], stg.at[1], sem)
        da.wait()
        db.wait()

        def _grp(g, carry2):
            for q in range(8):
                res = _blend_group(g, q, stg, yv, xv, 0)
                i_row = 4 * g + (q // 2)
                outF[i_row, pl.ds((q % 2) * 16, 16)] = res
            return carry2
        lax.fori_loop(0, 32, _grp, 0)
        pltpu.sync_copy(outF, out_hbm.at[m, k, h])
        return carry

    lax.fori_loop(0, XB_QPW, _q, 0)


_sc_xb = pl.kernel(
    _xb_body,
    out_type=jax.ShapeDtypeStruct((B * C, NPATCH, 4, P, JW), _f32),
    mesh=_MESH,
    compiler_params=pltpu.CompilerParams(needs_layout_passes=False),
    scratch_types=[
        pltpu.VMEM((4, JW), _i32),
        pltpu.VMEM((64, 128), _i32),
        pltpu.VMEM((2, JW, IMG_H), _f32),
        pltpu.VMEM((P, JW), _f32),
        pltpu.SemaphoreType.DMA,
    ],
)


GT_UNITS = NPATCH * 4
GT_T = (GT_UNITS + NW - 1) // NW
ZCHUNK_GT = 4096


def _gt_body(gtT_hbm, xpk_hbm, ypk_hbm, wtpk_hbm, accp_hbm,
             xv3, yv3, wt3, stg, valsS, zbuf, acc, sem, sem2):
    wid = _wid()
    sid = lax.axis_index("s")
    core = lax.axis_index("c")
    zf = jnp.zeros((16,), _f32)

    for t in range(GT_T):
        su = wid + NW * t

        @pl.when(su < GT_UNITS)
        def _():
            k = su // 4
            h = su % 4
            pltpu.sync_copy(xpk_hbm.at[k, h], xv3.at[t])
            pltpu.sync_copy(ypk_hbm.at[k, h], yv3.at[pl.ds(t * 64, 64)])
            pltpu.sync_copy(wtpk_hbm.at[k, h], wt3.at[t])

    def _zb(q, carry):
        zbuf[pl.ds(q * 16, 16)] = zf
        return carry
    lax.fori_loop(0, ZCHUNK_GT // 16, _zb, 0)

    def _batch(b, carry):
        for q in range(ACC_PER_TILE // ZCHUNK_GT):
            pltpu.sync_copy(zbuf, acc.at[pl.ds(sid * ACC_PER_TILE + q * ZCHUNK_GT, ZCHUNK_GT)])
        plsc.subcore_barrier()
        for t in range(GT_T):
            su = wid + NW * t

            @pl.when(su < GT_UNITS)
            def _():
                da = pltpu.async_copy(gtT_hbm.at[b].at[xv3.at[t, 0]], stg.at[0], sem)
                db = pltpu.async_copy(gtT_hbm.at[b].at[xv3.at[t, 1]], stg.at[1], sem)
                da.wait()
                db.wait()

                def _grp(g, carry2):
                    for q in range(8):
                        res = _blend_group(g, q, stg, yv3, xv3.at[t], t * 64)
                        w = plsc.bitcast(wt3[t, 32 + g, pl.ds(q * 16, 16)], _f32)
                        valsS[g, pl.ds(q * 16, 16)] = res * w
                    return carry2
                lax.fori_loop(0, 32, _grp, 0)

                descs = []
                for g in range(32):
                    descs.append(pltpu.async_copy(
                        valsS.at[g], acc.at[wt3.at[t, g]], sem2, add=True))
                for d in descs:
                    d.wait()
        plsc.subcore_barrier()
        for q in range(ACC_PER_TILE // ZCHUNK):
            off = sid * ACC_PER_TILE + q * ZCHUNK
            pltpu.sync_copy(acc.at[pl.ds(off, ZCHUNK)],
                            accp_hbm.at[core, b, pl.ds(off, ZCHUNK)])
        plsc.subcore_barrier()
        return carry

    lax.fori_loop(0, B, _batch, 0)


_sc_gt = pl.kernel(
    _gt_body,
    out_type=jax.ShapeDtypeStruct((NC, B, HW), _f32),
    mesh=_MESH,
    compiler_params=pltpu.CompilerParams(needs_layout_passes=False),
    scratch_types=[
        pltpu.VMEM((GT_T, 4, JW), _i32),
        pltpu.VMEM((GT_T * 64, 128), _i32),
        pltpu.VMEM((GT_T, 64, 128), _i32),
        pltpu.VMEM((2, JW, IMG_H), _f32),
        pltpu.VMEM((JW, 128), _f32),
        pltpu.VMEM((ZCHUNK_GT,), _f32),
        pltpu.VMEM_SHARED((HW,), _f32),
        pltpu.SemaphoreType.DMA,
        pltpu.SemaphoreType.DMA,
    ],
)


ROWCHUNK = 128


def _fin_body(accp_ref, cnt2_ref, out_ref):
    num = accp_ref[0, 0] + accp_ref[1, 0]
    den = jnp.maximum(cnt2_ref[0] + cnt2_ref[1], 1.0)
    out_ref[...] = (num / den)[None]


def _tc_fin(accp, cnt2p):
    nchunk = IMG_H // ROWCHUNK
    accp = accp.reshape(NC, B, IMG_H, IMG_W)
    cnt2p = cnt2p.reshape(NC, IMG_H, IMG_W)
    return pl.pallas_call(
        _fin_body,
        grid=(B, nchunk),
        in_specs=[
            pl.BlockSpec((NC, 1, ROWCHUNK, IMG_W), lambda b, j: (0, b, j, 0)),
            pl.BlockSpec((NC, ROWCHUNK, IMG_W), lambda b, j: (0, j, 0)),
        ],
        out_specs=pl.BlockSpec((1, ROWCHUNK, IMG_W), lambda b, j: (b, j, 0)),
        out_shape=jax.ShapeDtypeStruct((B, IMG_H, IMG_W), _f32),
    )(accp, cnt2p)


def _flatten_q(a):
    return (a.reshape(NPATCH, P, 4, JW).transpose(0, 2, 1, 3)
             .reshape(NPATCH, 4, JW, 128))


def _bits(a):
    return lax.bitcast_convert_type(a.astype(_f32), _i32)


def kernel(xb, gt, grid, x_idx, y_buf):
    g3 = grid.reshape(NPATCH, P, P, 2)
    gxk = g3[:, 0, :, 0]
    gyk = g3[..., 1]

    px = (gxk + 1.0) * 0.5 * (IMG_W - 1)
    x0 = jnp.floor(px)
    fx = px - x0
    x0i = x0.astype(_i32)
    x1i = x0i + 1
    wx0 = 1.0 - fx
    wx1 = jnp.where(x1i <= IMG_W - 1, fx, 0.0)
    x1c = jnp.clip(x1i, 0, IMG_W - 1)

    py = (gyk + 1.0) * 0.5 * (IMG_H - 1)
    y0 = jnp.floor(py)
    fy = (py - y0).astype(_f32)
    y0i = y0.astype(_i32)

    xpk = jnp.stack([x0i.reshape(NPATCH, 4, JW), x1c.reshape(NPATCH, 4, JW),
                     _bits(wx0.reshape(NPATCH, 4, JW)),
                     _bits(wx1.reshape(NPATCH, 4, JW))], axis=2)
    ypk = jnp.concatenate([_flatten_q(y0i), _bits(_flatten_q(fy))],
                          axis=2)

    xbT = xb.reshape(B * C, IMG_H, IMG_W).transpose(0, 2, 1)
    gtT = gt.transpose(0, 2, 1)

    w2304, tgt2304, cnt2p = _sc_pre(y_buf, x_idx)
    wtpk = jnp.concatenate([_flatten_q(tgt2304.reshape(NPATCH, P, P)),
                            _bits(_flatten_q(w2304.reshape(NPATCH, P, P)))],
                           axis=2)

    out1h = _sc_xb(xbT, xpk, ypk)
    out1 = (out1h.reshape(B, C, NPATCH, 4, P, JW)
                 .transpose(0, 1, 2, 4, 3, 5)
                 .reshape(B, C, NPATCH, P, P))

    accp = _sc_gt(gtT, xpk, ypk, wtpk)
    out2 = _tc_fin(accp, cnt2p)
    return out1, out2


# speedup vs baseline: 39.4630x; 1.0658x over previous
import functools

import jax
import jax.numpy as jnp
from jax import lax
from jax.experimental import pallas as pl
from jax.experimental.pallas import tpu as pltpu
from jax.experimental.pallas import tpu_sc as plsc

NPATCH = 18
P = 128
IMG_H = 512
IMG_W = 1024
B = 8
C = 3
R = NPATCH * P
HW = IMG_H * IMG_W
NC = 2
NS = 16
NW = NC * NS
JW = 32

_MESH = plsc.VectorSubcoreMesh(core_axis_name="c", subcore_axis_name="s")

_i32 = jnp.int32
_f32 = jnp.float32


def _wid():
    return lax.axis_index("s") * NC + lax.axis_index("c")


def _iota16():
    return lax.iota(_i32, 16)


ROWS_PER_W = R // NW
ROW_TILE = 8
N_TILES = ROWS_PER_W // ROW_TILE
ZCHUNK = 16384
ACC_PER_TILE = HW // NS


def _pre_body(ybuf_hbm, xidx_hbm, w_hbm, tgt_hbm, cnt2_hbm,
              ybrows, xirows, wrows, trows, cntbuf, idx2, ones128, zbuf,
              cnt2acc):
    wid = _wid()
    sid = lax.axis_index("s")
    core = lax.axis_index("c")
    ones = jnp.ones((16,), _f32)
    zf = jnp.zeros((16,), _f32)

    for q in range(8):
        ones128[pl.ds(q * 16, 16)] = ones

    def _zb(q, carry):
        zbuf[pl.ds(q * 16, 16)] = zf
        return carry
    lax.fori_loop(0, ZCHUNK // 16, _zb, 0)

    for q in range(ACC_PER_TILE // ZCHUNK):
        pltpu.sync_copy(zbuf, cnt2acc.at[pl.ds(sid * ACC_PER_TILE + q * ZCHUNK, ZCHUNK)])
    plsc.subcore_barrier()

    def _tile(t, carry):
        r0 = wid * ROWS_PER_W + t * ROW_TILE
        pltpu.sync_copy(ybuf_hbm.at[pl.ds(r0, ROW_TILE)], ybrows)
        pltpu.sync_copy(xidx_hbm.at[pl.ds(r0, ROW_TILE)], xirows)
        for rr in range(ROW_TILE):
            for q in range(IMG_W // 16):
                cntbuf[pl.ds(q * 16, 16)] = zf
            for q in range(P // 16):
                xi = xirows[rr, pl.ds(q * 16, 16)]
                plsc.addupdate_scatter(cntbuf, [xi], ones)
            rsel = jnp.full((16,), rr, _i32)
            for q in range(P // 16):
                xi = xirows[rr, pl.ds(q * 16, 16)]
                cnt = plsc.load_gather(cntbuf, [xi])
                wrows[rr, pl.ds(q * 16, 16)] = 1.0 / cnt
                yb = plsc.load_gather(ybrows, [rsel, xi])
                trows[rr, pl.ds(q * 16, 16)] = yb * IMG_W + xi
            for s in range(8):
                for q2 in range(8):
                    off = s * 128 + q2 * 16
                    yb = ybrows[rr, pl.ds(off, 16)]
                    idx2[s, pl.ds(q2 * 16, 16)] = yb * IMG_W + (off + _iota16())
            for s in range(8):
                pltpu.sync_copy(ones128, cnt2acc.at[idx2.at[s]], add=True)
        pltpu.sync_copy(wrows, w_hbm.at[pl.ds(r0, ROW_TILE)])
        pltpu.sync_copy(trows, tgt_hbm.at[pl.ds(r0, ROW_TILE)])
        return carry

    lax.fori_loop(0, N_TILES, _tile, 0)
    plsc.subcore_barrier()
    for q in range(ACC_PER_TILE // ZCHUNK):
        off = sid * ACC_PER_TILE + q * ZCHUNK
        pltpu.sync_copy(cnt2acc.at[pl.ds(off, ZCHUNK)],
                        cnt2_hbm.at[core, pl.ds(off, ZCHUNK)])


_sc_pre = pl.kernel(
    _pre_body,
    out_type=(
        jax.ShapeDtypeStruct((R, P), _f32),
        jax.ShapeDtypeStruct((R, P), _i32),
        jax.ShapeDtypeStruct((NC, HW), _f32),
    ),
    mesh=_MESH,
    compiler_params=pltpu.CompilerParams(needs_layout_passes=False),
    scratch_types=[
        pltpu.VMEM((ROW_TILE, IMG_W), _i32),
        pltpu.VMEM((ROW_TILE, P), _i32),
        pltpu.VMEM((ROW_TILE, P), _f32),
        pltpu.VMEM((ROW_TILE, P), _i32),
        pltpu.VMEM((IMG_W,), _f32),
        pltpu.VMEM((8, 128), _i32),
        pltpu.VMEM((128,), _f32),
        pltpu.VMEM((ZCHUNK,), _f32),
        pltpu.VMEM_SHARED((HW,), _f32),
    ],
)


def _blend_group(g, q, stg, yv, xv, yoff):
    col0 = q * 16
    jl0 = (q % 2) * 16
    y0 = yv[yoff + g, pl.ds(col0, 16)]
    fy = plsc.bitcast(yv[yoff + 32 + g, pl.ds(col0, 16)], _f32)
    y1 = jnp.minimum(y0 + 1, IMG_H - 1)
    wy1 = jnp.where(y0 <= IMG_H - 2, fy, 0.0)
    wy0 = 1.0 - fy
    jlv = jl0 + _iota16()
    z16 = jnp.zeros((16,), _i32)
    o16 = jnp.ones((16,), _i32)
    a00 = plsc.load_gather(stg, [z16, jlv, y0])
    a01 = plsc.load_gather(stg, [z16, jlv, y1])
    b00 = plsc.load_gather(stg, [o16, jlv, y0])
    b01 = plsc.load_gather(stg, [o16, jlv, y1])
    wxa = plsc.bitcast(xv[2, pl.ds(jl0, 16)], _f32)
    wxb = plsc.bitcast(xv[3, pl.ds(jl0, 16)], _f32)
    return (a00 * wy0 + a01 * wy1) * wxa + (b00 * wy0 + b01 * wy1) * wxb


XB_Q = B * C * NPATCH * 4
XB_QPW = XB_Q // NW
MM = B * C


def _xb_issue(imgT_hbm, xv, stg2, sems, m, buf):
    da = pltpu.async_copy(imgT_hbm.at[m].at[xv.at[0]], stg2.at[buf, 0], sems[buf])
    db = pltpu.async_copy(imgT_hbm.at[m].at[xv.at[1]], stg2.at[buf, 1], sems[buf])
    return da, db


def _xb_body(imgT_hbm, xpk_hbm, ypk_hbm, out_hbm,
             xv, yv, stg2, outF, semA, semB):
    wid = _wid()
    u0 = wid * XB_QPW
    kh_first = u0 // MM
    sems = (semA, semB)

    for seg in range(4):
        kh = kh_first + seg
        seg_lo = jnp.maximum(u0, kh * MM)
        seg_hi = jnp.minimum(u0 + XB_QPW, (kh + 1) * MM)

        @pl.when(seg_lo < seg_hi)
        def _():
            k = kh // 4
            h = kh % 4
            pltpu.sync_copy(xpk_hbm.at[k, h], xv)
            pltpu.sync_copy(ypk_hbm.at[k, h], yv)
            m_lo = seg_lo - kh * MM
            m_hi = seg_hi - kh * MM

            for par in range(2):
                @pl.when((m_lo & 1) == par)
                def _():
                    _xb_issue(imgT_hbm, xv, stg2, sems, m_lo, par)

            def _m(m, carry):
                pb = m & 1

                @pl.when(m + 1 < m_hi)
                def _():
                    for par in range(2):
                        @pl.when(pb == par)
                        def _():
                            _xb_issue(imgT_hbm, xv, stg2, sems, m + 1, 1 - par)

                for par in range(2):
                    @pl.when(pb == par)
                    def _():
                        da = pltpu.make_async_copy(
                            imgT_hbm.at[m].at[xv.at[0]], stg2.at[par, 0], sems[par])
                        db = pltpu.make_async_copy(
                            imgT_hbm.at[m].at[xv.at[1]], stg2.at[par, 1], sems[par])
                        da.wait()
                        db.wait()

                        def _grp(g, carry2):
                            for q in range(8):
                                res = _blend_group(g, q, stg2.at[par], yv, xv, 0)
                                i_row = 4 * g + (q // 2)
                                outF[i_row, pl.ds((q % 2) * 16, 16)] = res
                            return carry2
                        lax.fori_loop(0, 32, _grp, 0)
                pltpu.sync_copy(outF, out_hbm.at[m, k, h])
                return carry

            lax.fori_loop(m_lo, m_hi, _m, 0)


_sc_xb = pl.kernel(
    _xb_body,
    out_type=jax.ShapeDtypeStruct((B * C, NPATCH, 4, P, JW), _f32),
    mesh=_MESH,
    compiler_params=pltpu.CompilerParams(needs_layout_passes=False),
    scratch_types=[
        pltpu.VMEM((4, JW), _i32),
        pltpu.VMEM((64, 128), _i32),
        pltpu.VMEM((2, 2, JW, IMG_H), _f32),
        pltpu.VMEM((P, JW), _f32),
        pltpu.SemaphoreType.DMA,
        pltpu.SemaphoreType.DMA,
    ],
)


GT_UNITS = NPATCH * 4
GT_T = (GT_UNITS + NW - 1) // NW
ZCHUNK_GT = 4096


def _gt_body(gtT_hbm, xpk_hbm, ypk_hbm, wtpk_hbm, accp_hbm,
             xv3, yv3, wt3, stg, valsS, zbuf, acc, sem, sem2):
    wid = _wid()
    sid = lax.axis_index("s")
    core = lax.axis_index("c")
    zf = jnp.zeros((16,), _f32)

    for t in range(GT_T):
        su = wid + NW * t

        @pl.when(su < GT_UNITS)
        def _():
            k = su // 4
            h = su % 4
            pltpu.sync_copy(xpk_hbm.at[k, h], xv3.at[t])
            pltpu.sync_copy(ypk_hbm.at[k, h], yv3.at[pl.ds(t * 64, 64)])
            pltpu.sync_copy(wtpk_hbm.at[k, h], wt3.at[t])

    def _zb(q, carry):
        zbuf[pl.ds(q * 16, 16)] = zf
        return carry
    lax.fori_loop(0, ZCHUNK_GT // 16, _zb, 0)

    def _batch(b, carry):
        for q in range(ACC_PER_TILE // ZCHUNK_GT):
            pltpu.sync_copy(zbuf, acc.at[pl.ds(sid * ACC_PER_TILE + q * ZCHUNK_GT, ZCHUNK_GT)])
        plsc.subcore_barrier()
        for t in range(GT_T):
            su = wid + NW * t

            @pl.when(su < GT_UNITS)
            def _():
                da = pltpu.async_copy(gtT_hbm.at[b].at[xv3.at[t, 0]], stg.at[0], sem)
                db = pltpu.async_copy(gtT_hbm.at[b].at[xv3.at[t, 1]], stg.at[1], sem)
                da.wait()
                db.wait()

                def _grp(g, carry2):
                    for q in range(8):
                        res = _blend_group(g, q, stg, yv3, xv3.at[t], t * 64)
                        w = plsc.bitcast(wt3[t, 32 + g, pl.ds(q * 16, 16)], _f32)
                        valsS[g, pl.ds(q * 16, 16)] = res * w
                    return carry2
                lax.fori_loop(0, 32, _grp, 0)

                descs = []
                for g in range(32):
                    descs.append(pltpu.async_copy(
                        valsS.at[g], acc.at[wt3.at[t, g]], sem2, add=True))
                for d in descs:
                    d.wait()
        plsc.subcore_barrier()
        for q in range(ACC_PER_TILE // ZCHUNK):
            off = sid * ACC_PER_TILE + q * ZCHUNK
            pltpu.sync_copy(acc.at[pl.ds(off, ZCHUNK)],
                            accp_hbm.at[core, b, pl.ds(off, ZCHUNK)])
        plsc.subcore_barrier()
        return carry

    lax.fori_loop(0, B, _batch, 0)


_sc_gt = pl.kernel(
    _gt_body,
    out_type=jax.ShapeDtypeStruct((NC, B, HW), _f32),
    mesh=_MESH,
    compiler_params=pltpu.CompilerParams(needs_layout_passes=False),
    scratch_types=[
        pltpu.VMEM((GT_T, 4, JW), _i32),
        pltpu.VMEM((GT_T * 64, 128), _i32),
        pltpu.VMEM((GT_T, 64, 128), _i32),
        pltpu.VMEM((2, JW, IMG_H), _f32),
        pltpu.VMEM((JW, 128), _f32),
        pltpu.VMEM((ZCHUNK_GT,), _f32),
        pltpu.VMEM_SHARED((HW,), _f32),
        pltpu.SemaphoreType.DMA,
        pltpu.SemaphoreType.DMA,
    ],
)


ROWCHUNK = 128


def _fin_body(accp_ref, cnt2_ref, out_ref):
    num = accp_ref[0, 0] + accp_ref[1, 0]
    den = jnp.maximum(cnt2_ref[0] + cnt2_ref[1], 1.0)
    out_ref[...] = (num / den)[None]


def _tc_fin(accp, cnt2p):
    nchunk = IMG_H // ROWCHUNK
    accp = accp.reshape(NC, B, IMG_H, IMG_W)
    cnt2p = cnt2p.reshape(NC, IMG_H, IMG_W)
    return pl.pallas_call(
        _fin_body,
        grid=(B, nchunk),
        in_specs=[
            pl.BlockSpec((NC, 1, ROWCHUNK, IMG_W), lambda b, j: (0, b, j, 0)),
            pl.BlockSpec((NC, ROWCHUNK, IMG_W), lambda b, j: (0, j, 0)),
        ],
        out_specs=pl.BlockSpec((1, ROWCHUNK, IMG_W), lambda b, j: (b, j, 0)),
        out_shape=jax.ShapeDtypeStruct((B, IMG_H, IMG_W), _f32),
    )(accp, cnt2p)


def _flatten_q(a):
    return (a.reshape(NPATCH, P, 4, JW).transpose(0, 2, 1, 3)
             .reshape(NPATCH, 4, JW, 128))


def _bits(a):
    return lax.bitcast_convert_type(a.astype(_f32), _i32)


def kernel(xb, gt, grid, x_idx, y_buf):
    g3 = grid.reshape(NPATCH, P, P, 2)
    gxk = g3[:, 0, :, 0]
    gyk = g3[..., 1]

    px = (gxk + 1.0) * 0.5 * (IMG_W - 1)
    x0 = jnp.floor(px)
    fx = px - x0
    x0i = x0.astype(_i32)
    x1i = x0i + 1
    wx0 = 1.0 - fx
    wx1 = jnp.where(x1i <= IMG_W - 1, fx, 0.0)
    x1c = jnp.clip(x1i, 0, IMG_W - 1)

    py = (gyk + 1.0) * 0.5 * (IMG_H - 1)
    y0 = jnp.floor(py)
    fy = (py - y0).astype(_f32)
    y0i = y0.astype(_i32)

    xpk = jnp.stack([x0i.reshape(NPATCH, 4, JW), x1c.reshape(NPATCH, 4, JW),
                     _bits(wx0.reshape(NPATCH, 4, JW)),
                     _bits(wx1.reshape(NPATCH, 4, JW))], axis=2)
    ypk = jnp.concatenate([_flatten_q(y0i), _bits(_flatten_q(fy))],
                          axis=2)

    xbT = xb.reshape(B * C, IMG_H, IMG_W).transpose(0, 2, 1)
    gtT = gt.transpose(0, 2, 1)

    w2304, tgt2304, cnt2p = _sc_pre(y_buf, x_idx)
    wtpk = jnp.concatenate([_flatten_q(tgt2304.reshape(NPATCH, P, P)),
                            _bits(_flatten_q(w2304.reshape(NPATCH, P, P)))],
                           axis=2)

    out1h = _sc_xb(xbT, xpk, ypk)
    out1 = (out1h.reshape(B, C, NPATCH, 4, P, JW)
                 .transpose(0, 1, 2, 4, 3, 5)
                 .reshape(B, C, NPATCH, P, P))

    accp = _sc_gt(gtT, xpk, ypk, wtpk)
    out2 = _tc_fin(accp, cnt2p)
    return out1, out2


# speedup vs baseline: 40.9932x; 1.0388x over previous
import functools

import jax
import jax.numpy as jnp
from jax import lax
from jax.experimental import pallas as pl
from jax.experimental.pallas import tpu as pltpu
from jax.experimental.pallas import tpu_sc as plsc

NPATCH = 18
P = 128
IMG_H = 512
IMG_W = 1024
B = 8
C = 3
R = NPATCH * P
HW = IMG_H * IMG_W
NC = 2
NS = 16
NW = NC * NS
JW = 32

_MESH = plsc.VectorSubcoreMesh(core_axis_name="c", subcore_axis_name="s")

_i32 = jnp.int32
_f32 = jnp.float32


def _wid():
    return lax.axis_index("s") * NC + lax.axis_index("c")


def _iota16():
    return lax.iota(_i32, 16)


ROWS_PER_W = R // NW
ROW_TILE = 8
N_TILES = ROWS_PER_W // ROW_TILE
ZCHUNK = 16384
ACC_PER_TILE = HW // NS


def _pre_body(ybuf_hbm, xidx_hbm, w_hbm, tgt_hbm, cnt2_hbm,
              ybrows, xirows, wrows, trows, cntbuf, idx2, ones128, zbuf,
              cnt2acc):
    wid = _wid()
    sid = lax.axis_index("s")
    core = lax.axis_index("c")
    ones = jnp.ones((16,), _f32)
    zf = jnp.zeros((16,), _f32)

    for q in range(8):
        ones128[pl.ds(q * 16, 16)] = ones

    def _zb(q, carry):
        zbuf[pl.ds(q * 16, 16)] = zf
        return carry
    lax.fori_loop(0, ZCHUNK // 16, _zb, 0)

    for q in range(ACC_PER_TILE // ZCHUNK):
        pltpu.sync_copy(zbuf, cnt2acc.at[pl.ds(sid * ACC_PER_TILE + q * ZCHUNK, ZCHUNK)])
    plsc.subcore_barrier()

    def _tile(t, carry):
        r0 = wid * ROWS_PER_W + t * ROW_TILE
        pltpu.sync_copy(ybuf_hbm.at[pl.ds(r0, ROW_TILE)], ybrows)
        pltpu.sync_copy(xidx_hbm.at[pl.ds(r0, ROW_TILE)], xirows)
        for rr in range(ROW_TILE):
            for q in range(IMG_W // 16):
                cntbuf[pl.ds(q * 16, 16)] = zf
            for q in range(P // 16):
                xi = xirows[rr, pl.ds(q * 16, 16)]
                plsc.addupdate_scatter(cntbuf, [xi], ones)
            rsel = jnp.full((16,), rr, _i32)
            for q in range(P // 16):
                xi = xirows[rr, pl.ds(q * 16, 16)]
                cnt = plsc.load_gather(cntbuf, [xi])
                wrows[rr, pl.ds(q * 16, 16)] = 1.0 / cnt
                yb = plsc.load_gather(ybrows, [rsel, xi])
                trows[rr, pl.ds(q * 16, 16)] = yb * IMG_W + xi
            for s in range(8):
                for q2 in range(8):
                    off = s * 128 + q2 * 16
                    yb = ybrows[rr, pl.ds(off, 16)]
                    idx2[s, pl.ds(q2 * 16, 16)] = yb * IMG_W + (off + _iota16())
            for s in range(8):
                pltpu.sync_copy(ones128, cnt2acc.at[idx2.at[s]], add=True)
        pltpu.sync_copy(wrows, w_hbm.at[pl.ds(r0, ROW_TILE)])
        pltpu.sync_copy(trows, tgt_hbm.at[pl.ds(r0, ROW_TILE)])
        return carry

    lax.fori_loop(0, N_TILES, _tile, 0)
    plsc.subcore_barrier()
    for q in range(ACC_PER_TILE // ZCHUNK):
        off = sid * ACC_PER_TILE + q * ZCHUNK
        pltpu.sync_copy(cnt2acc.at[pl.ds(off, ZCHUNK)],
                        cnt2_hbm.at[core, pl.ds(off, ZCHUNK)])


_sc_pre = pl.kernel(
    _pre_body,
    out_type=(
        jax.ShapeDtypeStruct((R, P), _f32),
        jax.ShapeDtypeStruct((R, P), _i32),
        jax.ShapeDtypeStruct((NC, HW), _f32),
    ),
    mesh=_MESH,
    compiler_params=pltpu.CompilerParams(needs_layout_passes=False),
    scratch_types=[
        pltpu.VMEM((ROW_TILE, IMG_W), _i32),
        pltpu.VMEM((ROW_TILE, P), _i32),
        pltpu.VMEM((ROW_TILE, P), _f32),
        pltpu.VMEM((ROW_TILE, P), _i32),
        pltpu.VMEM((IMG_W,), _f32),
        pltpu.VMEM((8, 128), _i32),
        pltpu.VMEM((128,), _f32),
        pltpu.VMEM((ZCHUNK,), _f32),
        pltpu.VMEM_SHARED((HW,), _f32),
    ],
)


def _blend_group(g, q, stg, yv, xv, yoff):
    col0 = q * 16
    jl0 = (q % 2) * 16
    y0 = yv[yoff + g, pl.ds(col0, 16)]
    fy = plsc.bitcast(yv[yoff + 32 + g, pl.ds(col0, 16)], _f32)
    y1 = jnp.minimum(y0 + 1, IMG_H - 1)
    wy1 = jnp.where(y0 <= IMG_H - 2, fy, 0.0)
    wy0 = 1.0 - fy
    jlv = jl0 + _iota16()
    z16 = jnp.zeros((16,), _i32)
    o16 = jnp.ones((16,), _i32)
    a00 = plsc.load_gather(stg, [z16, jlv, y0])
    a01 = plsc.load_gather(stg, [z16, jlv, y1])
    b00 = plsc.load_gather(stg, [o16, jlv, y0])
    b01 = plsc.load_gather(stg, [o16, jlv, y1])
    wxa = plsc.bitcast(xv[2, pl.ds(jl0, 16)], _f32)
    wxb = plsc.bitcast(xv[3, pl.ds(jl0, 16)], _f32)
    return (a00 * wy0 + a01 * wy1) * wxa + (b00 * wy0 + b01 * wy1) * wxb


XB_Q = B * C * NPATCH * 4
XB_QPW = XB_Q // NW
MM = B * C


def _xb_issue(imgT_hbm, xv, stg2, sems, m, buf):
    da = pltpu.async_copy(imgT_hbm.at[m].at[xv.at[0]],
                          stg2.at[buf, pl.ds(0, JW)], sems[buf])
    db = pltpu.async_copy(imgT_hbm.at[m].at[xv.at[1]],
                          stg2.at[buf, pl.ds(JW, JW)], sems[buf])
    return da, db


def _xb_body(imgT_hbm, xpk_hbm, ypk4_hbm, out_hbm,
             xv, yv, stg2, outF, semA, semB):
    wid = _wid()
    u0 = wid * XB_QPW
    kh_first = u0 // MM
    sems = (semA, semB)

    for seg in range(4):
        kh = kh_first + seg
        seg_lo = jnp.maximum(u0, kh * MM)
        seg_hi = jnp.minimum(u0 + XB_QPW, (kh + 1) * MM)

        @pl.when(seg_lo < seg_hi)
        def _():
            k = kh // 4
            h = kh % 4
            pltpu.sync_copy(xpk_hbm.at[k, h], xv)
            pltpu.sync_copy(ypk4_hbm.at[k, h], yv)
            m_lo = seg_lo - kh * MM
            m_hi = seg_hi - kh * MM
            jA = (_iota16(), _iota16() + 16)
            jB = (jA[0] + JW, jA[1] + JW)
            wxa = (plsc.bitcast(xv[2, pl.ds(0, 16)], _f32),
                   plsc.bitcast(xv[2, pl.ds(16, 16)], _f32))
            wxb = (plsc.bitcast(xv[3, pl.ds(0, 16)], _f32),
                   plsc.bitcast(xv[3, pl.ds(16, 16)], _f32))

            for par in range(2):
                @pl.when((m_lo & 1) == par)
                def _():
                    _xb_issue(imgT_hbm, xv, stg2, sems, m_lo, par)

            def _m(m, carry):
                pb = m & 1

                @pl.when(m + 1 < m_hi)
                def _():
                    for par in range(2):
                        @pl.when(pb == par)
                        def _():
                            _xb_issue(imgT_hbm, xv, stg2, sems, m + 1, 1 - par)

                for par in range(2):
                    @pl.when(pb == par)
                    def _():
                        pltpu.make_async_copy(
                            imgT_hbm.at[m].at[xv.at[0]],
                            stg2.at[par, pl.ds(0, JW)], sems[par]).wait()
                        pltpu.make_async_copy(
                            imgT_hbm.at[m].at[xv.at[1]],
                            stg2.at[par, pl.ds(JW, JW)], sems[par]).wait()
                        stg = stg2.at[par]

                        def _grp(g, carry2):
                            for q in range(8):
                                col0 = q * 16
                                pq = q % 2
                                y0 = yv[0, g, pl.ds(col0, 16)]
                                y1 = yv[1, g, pl.ds(col0, 16)]
                                wy0 = plsc.bitcast(yv[2, g, pl.ds(col0, 16)], _f32)
                                wy1 = plsc.bitcast(yv[3, g, pl.ds(col0, 16)], _f32)
                                a00 = plsc.load_gather(stg, [jA[pq], y0])
                                a01 = plsc.load_gather(stg, [jA[pq], y1])
                                b00 = plsc.load_gather(stg, [jB[pq], y0])
                                b01 = plsc.load_gather(stg, [jB[pq], y1])
                                res = ((a00 * wy0 + a01 * wy1) * wxa[pq]
                                       + (b00 * wy0 + b01 * wy1) * wxb[pq])
                                i_row = 4 * g + (q // 2)
                                outF[i_row, pl.ds(pq * 16, 16)] = res
                            return carry2
                        lax.fori_loop(0, 32, _grp, 0)
                pltpu.sync_copy(outF, out_hbm.at[m, k, h])
                return carry

            lax.fori_loop(m_lo, m_hi, _m, 0)


_sc_xb = pl.kernel(
    _xb_body,
    out_type=jax.ShapeDtypeStruct((B * C, NPATCH, 4, P, JW), _f32),
    mesh=_MESH,
    compiler_params=pltpu.CompilerParams(needs_layout_passes=False),
    scratch_types=[
        pltpu.VMEM((4, JW), _i32),
        pltpu.VMEM((4, JW, 128), _i32),
        pltpu.VMEM((2, 2 * JW, IMG_H), _f32),
        pltpu.VMEM((P, JW), _f32),
        pltpu.SemaphoreType.DMA,
        pltpu.SemaphoreType.DMA,
    ],
)


GT_UNITS = NPATCH * 4
GT_T = (GT_UNITS + NW - 1) // NW
ZCHUNK_GT = 4096


def _gt_body(gtT_hbm, xpk_hbm, ypk_hbm, wtpk_hbm, accp_hbm,
             xv3, yv3, wt3, stg, valsS, zbuf, acc, sem, sem2):
    wid = _wid()
    sid = lax.axis_index("s")
    core = lax.axis_index("c")
    zf = jnp.zeros((16,), _f32)

    for t in range(GT_T):
        su = wid + NW * t

        @pl.when(su < GT_UNITS)
        def _():
            k = su // 4
            h = su % 4
            pltpu.sync_copy(xpk_hbm.at[k, h], xv3.at[t])
            pltpu.sync_copy(ypk_hbm.at[k, h], yv3.at[pl.ds(t * 64, 64)])
            pltpu.sync_copy(wtpk_hbm.at[k, h], wt3.at[t])

    def _zb(q, carry):
        zbuf[pl.ds(q * 16, 16)] = zf
        return carry
    lax.fori_loop(0, ZCHUNK_GT // 16, _zb, 0)

    def _batch(b, carry):
        for q in range(ACC_PER_TILE // ZCHUNK_GT):
            pltpu.sync_copy(zbuf, acc.at[pl.ds(sid * ACC_PER_TILE + q * ZCHUNK_GT, ZCHUNK_GT)])
        plsc.subcore_barrier()
        for t in range(GT_T):
            su = wid + NW * t

            @pl.when(su < GT_UNITS)
            def _():
                da = pltpu.async_copy(gtT_hbm.at[b].at[xv3.at[t, 0]], stg.at[0], sem)
                db = pltpu.async_copy(gtT_hbm.at[b].at[xv3.at[t, 1]], stg.at[1], sem)
                da.wait()
                db.wait()

                def _grp(g, carry2):
                    for q in range(8):
                        res = _blend_group(g, q, stg, yv3, xv3.at[t], t * 64)
                        w = plsc.bitcast(wt3[t, 32 + g, pl.ds(q * 16, 16)], _f32)
                        valsS[g, pl.ds(q * 16, 16)] = res * w
                    return carry2
                lax.fori_loop(0, 32, _grp, 0)

                descs = []
                for g in range(32):
                    descs.append(pltpu.async_copy(
                        valsS.at[g], acc.at[wt3.at[t, g]], sem2, add=True))
                for d in descs:
                    d.wait()
        plsc.subcore_barrier()
        for q in range(ACC_PER_TILE // ZCHUNK):
            off = sid * ACC_PER_TILE + q * ZCHUNK
            pltpu.sync_copy(acc.at[pl.ds(off, ZCHUNK)],
                            accp_hbm.at[core, b, pl.ds(off, ZCHUNK)])
        plsc.subcore_barrier()
        return carry

    lax.fori_loop(0, B, _batch, 0)


_sc_gt = pl.kernel(
    _gt_body,
    out_type=jax.ShapeDtypeStruct((NC, B, HW), _f32),
    mesh=_MESH,
    compiler_params=pltpu.CompilerParams(needs_layout_passes=False),
    scratch_types=[
        pltpu.VMEM((GT_T, 4, JW), _i32),
        pltpu.VMEM((GT_T * 64, 128), _i32),
        pltpu.VMEM((GT_T, 64, 128), _i32),
        pltpu.VMEM((2, JW, IMG_H), _f32),
        pltpu.VMEM((JW, 128), _f32),
        pltpu.VMEM((ZCHUNK_GT,), _f32),
        pltpu.VMEM_SHARED((HW,), _f32),
        pltpu.SemaphoreType.DMA,
        pltpu.SemaphoreType.DMA,
    ],
)


ROWCHUNK = 128


def _fin_body(accp_ref, cnt2_ref, out_ref):
    num = accp_ref[0, 0] + accp_ref[1, 0]
    den = jnp.maximum(cnt2_ref[0] + cnt2_ref[1], 1.0)
    out_ref[...] = (num / den)[None]


def _tc_fin(accp, cnt2p):
    nchunk = IMG_H // ROWCHUNK
    accp = accp.reshape(NC, B, IMG_H, IMG_W)
    cnt2p = cnt2p.reshape(NC, IMG_H, IMG_W)
    return pl.pallas_call(
        _fin_body,
        grid=(B, nchunk),
        in_specs=[
            pl.BlockSpec((NC, 1, ROWCHUNK, IMG_W), lambda b, j: (0, b, j, 0)),
            pl.BlockSpec((NC, ROWCHUNK, IMG_W), lambda b, j: (0, j, 0)),
        ],
        out_specs=pl.BlockSpec((1, ROWCHUNK, IMG_W), lambda b, j: (b, j, 0)),
        out_shape=jax.ShapeDtypeStruct((B, IMG_H, IMG_W), _f32),
    )(accp, cnt2p)


def _flatten_q(a):
    return (a.reshape(NPATCH, P, 4, JW).transpose(0, 2, 1, 3)
             .reshape(NPATCH, 4, JW, 128))


def _bits(a):
    return lax.bitcast_convert_type(a.astype(_f32), _i32)


def kernel(xb, gt, grid, x_idx, y_buf):
    g3 = grid.reshape(NPATCH, P, P, 2)
    gxk = g3[:, 0, :, 0]
    gyk = g3[..., 1]

    px = (gxk + 1.0) * 0.5 * (IMG_W - 1)
    x0 = jnp.floor(px)
    fx = px - x0
    x0i = x0.astype(_i32)
    x1i = x0i + 1
    wx0 = 1.0 - fx
    wx1 = jnp.where(x1i <= IMG_W - 1, fx, 0.0)
    x1c = jnp.clip(x1i, 0, IMG_W - 1)

    py = (gyk + 1.0) * 0.5 * (IMG_H - 1)
    y0 = jnp.floor(py)
    fy = (py - y0).astype(_f32)
    y0i = y0.astype(_i32)

    xpk = jnp.stack([x0i.reshape(NPATCH, 4, JW), x1c.reshape(NPATCH, 4, JW),
                     _bits(wx0.reshape(NPATCH, 4, JW)),
                     _bits(wx1.reshape(NPATCH, 4, JW))], axis=2)
    ypk = jnp.concatenate([_flatten_q(y0i), _bits(_flatten_q(fy))],
                          axis=2)
    y1c = jnp.minimum(y0i + 1, IMG_H - 1)
    wy0a = 1.0 - fy
    wy1a = jnp.where(y0i <= IMG_H - 2, fy, 0.0)
    ypk4 = jnp.stack([_flatten_q(y0i), _flatten_q(y1c),
                      _bits(_flatten_q(wy0a)), _bits(_flatten_q(wy1a))],
                     axis=2)

    xbT = xb.reshape(B * C, IMG_H, IMG_W).transpose(0, 2, 1)
    gtT = gt.transpose(0, 2, 1)

    w2304, tgt2304, cnt2p = _sc_pre(y_buf, x_idx)
    wtpk = jnp.concatenate([_flatten_q(tgt2304.reshape(NPATCH, P, P)),
                            _bits(_flatten_q(w2304.reshape(NPATCH, P, P)))],
                           axis=2)

    out1h = _sc_xb(xbT, xpk, ypk4)
    out1 = (out1h.reshape(B, C, NPATCH, 4, P, JW)
                 .transpose(0, 1, 2, 4, 3, 5)
                 .reshape(B, C, NPATCH, P, P))

    accp = _sc_gt(gtT, xpk, ypk, wtpk)
    out2 = _tc_fin(accp, cnt2p)
    return out1, out2


# speedup vs baseline: 42.0934x; 1.0268x over previous
import functools

import jax
import jax.numpy as jnp
from jax import lax
from jax.experimental import pallas as pl
from jax.experimental.pallas import tpu as pltpu
from jax.experimental.pallas import tpu_sc as plsc

NPATCH = 18
P = 128
IMG_H = 512
IMG_W = 1024
B = 8
C = 3
R = NPATCH * P
HW = IMG_H * IMG_W
NC = 2
NS = 16
NW = NC * NS
JW = 32

_MESH = plsc.VectorSubcoreMesh(core_axis_name="c", subcore_axis_name="s")

_i32 = jnp.int32
_f32 = jnp.float32


def _wid():
    return lax.axis_index("s") * NC + lax.axis_index("c")


def _iota16():
    return lax.iota(_i32, 16)


ROWS_PER_W = R // NW
ROW_TILE = 8
N_TILES = ROWS_PER_W // ROW_TILE
ZCHUNK = 16384
ACC_PER_TILE = HW // NS


def _pre_body(ybuf_hbm, xidx_hbm, w_hbm, tgt_hbm, cnt2_hbm,
              ybrows, xirows, wrows, trows, cntbuf, idx2, ones128, zbuf,
              cnt2acc):
    wid = _wid()
    sid = lax.axis_index("s")
    core = lax.axis_index("c")
    ones = jnp.ones((16,), _f32)
    zf = jnp.zeros((16,), _f32)

    for q in range(8):
        ones128[pl.ds(q * 16, 16)] = ones

    def _zb(q, carry):
        zbuf[pl.ds(q * 16, 16)] = zf
        return carry
    lax.fori_loop(0, ZCHUNK // 16, _zb, 0)

    for q in range(ACC_PER_TILE // ZCHUNK):
        pltpu.sync_copy(zbuf, cnt2acc.at[pl.ds(sid * ACC_PER_TILE + q * ZCHUNK, ZCHUNK)])
    plsc.subcore_barrier()

    def _tile(t, carry):
        r0 = wid * ROWS_PER_W + t * ROW_TILE
        pltpu.sync_copy(ybuf_hbm.at[pl.ds(r0, ROW_TILE)], ybrows)
        pltpu.sync_copy(xidx_hbm.at[pl.ds(r0, ROW_TILE)], xirows)
        for rr in range(ROW_TILE):
            for q in range(IMG_W // 16):
                cntbuf[pl.ds(q * 16, 16)] = zf
            for q in range(P // 16):
                xi = xirows[rr, pl.ds(q * 16, 16)]
                plsc.addupdate_scatter(cntbuf, [xi], ones)
            rsel = jnp.full((16,), rr, _i32)
            for q in range(P // 16):
                xi = xirows[rr, pl.ds(q * 16, 16)]
                cnt = plsc.load_gather(cntbuf, [xi])
                wrows[rr, pl.ds(q * 16, 16)] = 1.0 / cnt
                yb = plsc.load_gather(ybrows, [rsel, xi])
                trows[rr, pl.ds(q * 16, 16)] = yb * IMG_W + xi
            for s in range(8):
                for q2 in range(8):
                    off = s * 128 + q2 * 16
                    yb = ybrows[rr, pl.ds(off, 16)]
                    idx2[s, pl.ds(q2 * 16, 16)] = yb * IMG_W + (off + _iota16())
            for s in range(8):
                pltpu.sync_copy(ones128, cnt2acc.at[idx2.at[s]], add=True)
        pltpu.sync_copy(wrows, w_hbm.at[pl.ds(r0, ROW_TILE)])
        pltpu.sync_copy(trows, tgt_hbm.at[pl.ds(r0, ROW_TILE)])
        return carry

    lax.fori_loop(0, N_TILES, _tile, 0)
    plsc.subcore_barrier()
    for q in range(ACC_PER_TILE // ZCHUNK):
        off = sid * ACC_PER_TILE + q * ZCHUNK
        pltpu.sync_copy(cnt2acc.at[pl.ds(off, ZCHUNK)],
                        cnt2_hbm.at[core, pl.ds(off, ZCHUNK)])


_sc_pre = pl.kernel(
    _pre_body,
    out_type=(
        jax.ShapeDtypeStruct((R, P), _f32),
        jax.ShapeDtypeStruct((R, P), _i32),
        jax.ShapeDtypeStruct((NC, HW), _f32),
    ),
    mesh=_MESH,
    compiler_params=pltpu.CompilerParams(needs_layout_passes=False),
    scratch_types=[
        pltpu.VMEM((ROW_TILE, IMG_W), _i32),
        pltpu.VMEM((ROW_TILE, P), _i32),
        pltpu.VMEM((ROW_TILE, P), _f32),
        pltpu.VMEM((ROW_TILE, P), _i32),
        pltpu.VMEM((IMG_W,), _f32),
        pltpu.VMEM((8, 128), _i32),
        pltpu.VMEM((128,), _f32),
        pltpu.VMEM((ZCHUNK,), _f32),
        pltpu.VMEM_SHARED((HW,), _f32),
    ],
)


def _blend_group(g, q, stg, yv, xv, yoff):
    col0 = q * 16
    jl0 = (q % 2) * 16
    y0 = yv[yoff + g, pl.ds(col0, 16)]
    fy = plsc.bitcast(yv[yoff + 32 + g, pl.ds(col0, 16)], _f32)
    y1 = jnp.minimum(y0 + 1, IMG_H - 1)
    wy1 = jnp.where(y0 <= IMG_H - 2, fy, 0.0)
    wy0 = 1.0 - fy
    jlv = jl0 + _iota16()
    z16 = jnp.zeros((16,), _i32)
    o16 = jnp.ones((16,), _i32)
    a00 = plsc.load_gather(stg, [z16, jlv, y0])
    a01 = plsc.load_gather(stg, [z16, jlv, y1])
    b00 = plsc.load_gather(stg, [o16, jlv, y0])
    b01 = plsc.load_gather(stg, [o16, jlv, y1])
    wxa = plsc.bitcast(xv[2, pl.ds(jl0, 16)], _f32)
    wxb = plsc.bitcast(xv[3, pl.ds(jl0, 16)], _f32)
    return (a00 * wy0 + a01 * wy1) * wxa + (b00 * wy0 + b01 * wy1) * wxb


XB_Q = B * C * NPATCH * 4
XB_QPW = XB_Q // NW
MM = B * C


def _xb_issue(imgT_hbm, xv, stg2, sems, m, buf):
    da = pltpu.async_copy(imgT_hbm.at[m].at[xv.at[0]],
                          stg2.at[buf, pl.ds(0, JW)], sems[buf])
    db = pltpu.async_copy(imgT_hbm.at[m].at[xv.at[1]],
                          stg2.at[buf, pl.ds(JW, JW)], sems[buf])
    return da, db


def _xb_body(imgT_hbm, xpk_hbm, ypk4_hbm, out_hbm,
             xv, yv, stg2, outF2, semA, semB, semW0, semW1):
    wid = _wid()
    u0 = wid * XB_QPW
    kh_first = u0 // MM
    sems = (semA, semB)
    semsW = (semW0, semW1)

    for seg in range(4):
        kh = kh_first + seg
        seg_lo = jnp.maximum(u0, kh * MM)
        seg_hi = jnp.minimum(u0 + XB_QPW, (kh + 1) * MM)

        @pl.when(seg_lo < seg_hi)
        def _():
            k = kh // 4
            h = kh % 4
            pltpu.sync_copy(xpk_hbm.at[k, h], xv)
            pltpu.sync_copy(ypk4_hbm.at[k, h], yv)
            m_lo = seg_lo - kh * MM
            m_hi = seg_hi - kh * MM
            jA = (_iota16(), _iota16() + 16)
            jB = (jA[0] + JW, jA[1] + JW)
            wxa = (plsc.bitcast(xv[2, pl.ds(0, 16)], _f32),
                   plsc.bitcast(xv[2, pl.ds(16, 16)], _f32))
            wxb = (plsc.bitcast(xv[3, pl.ds(0, 16)], _f32),
                   plsc.bitcast(xv[3, pl.ds(16, 16)], _f32))

            for par in range(2):
                @pl.when((m_lo & 1) == par)
                def _():
                    _xb_issue(imgT_hbm, xv, stg2, sems, m_lo, par)

            def _m(m, carry):
                pb = m & 1

                @pl.when(m + 1 < m_hi)
                def _():
                    for par in range(2):
                        @pl.when(pb == par)
                        def _():
                            _xb_issue(imgT_hbm, xv, stg2, sems, m + 1, 1 - par)

                for par in range(2):
                    @pl.when(pb == par)
                    def _():
                        @pl.when(m - 2 >= m_lo)
                        def _():
                            pltpu.make_async_copy(
                                outF2.at[par], out_hbm.at[m - 2, k, h],
                                semsW[par]).wait()
                        pltpu.make_async_copy(
                            imgT_hbm.at[m].at[xv.at[0]],
                            stg2.at[par, pl.ds(0, JW)], sems[par]).wait()
                        pltpu.make_async_copy(
                            imgT_hbm.at[m].at[xv.at[1]],
                            stg2.at[par, pl.ds(JW, JW)], sems[par]).wait()
                        stg = stg2.at[par]
                        outF = outF2.at[par]

                        def _grp(g, carry2):
                            for q in range(8):
                                col0 = q * 16
                                pq = q % 2
                                y0 = yv[0, g, pl.ds(col0, 16)]
                                y1 = yv[1, g, pl.ds(col0, 16)]
                                wy0 = plsc.bitcast(yv[2, g, pl.ds(col0, 16)], _f32)
                                wy1 = plsc.bitcast(yv[3, g, pl.ds(col0, 16)], _f32)
                                a00 = plsc.load_gather(stg, [jA[pq], y0])
                                a01 = plsc.load_gather(stg, [jA[pq], y1])
                                b00 = plsc.load_gather(stg, [jB[pq], y0])
                                b01 = plsc.load_gather(stg, [jB[pq], y1])
                                res = ((a00 * wy0 + a01 * wy1) * wxa[pq]
                                       + (b00 * wy0 + b01 * wy1) * wxb[pq])
                                i_row = 4 * g + (q // 2)
                                outF[i_row, pl.ds(pq * 16, 16)] = res
                            return carry2
                        lax.fori_loop(0, 32, _grp, 0)
                        pltpu.async_copy(outF2.at[par], out_hbm.at[m, k, h],
                                         semsW[par])
                return carry

            lax.fori_loop(m_lo, m_hi, _m, 0)
            for par in range(2):
                for back in (2, 1):
                    d = m_hi - back

                    @pl.when(jnp.logical_and(d >= m_lo, (d & 1) == par))
                    def _():
                        pltpu.make_async_copy(
                            outF2.at[par], out_hbm.at[d, k, h],
                            semsW[par]).wait()


_sc_xb = pl.kernel(
    _xb_body,
    out_type=jax.ShapeDtypeStruct((B * C, NPATCH, 4, P, JW), _f32),
    mesh=_MESH,
    compiler_params=pltpu.CompilerParams(needs_layout_passes=False),
    scratch_types=[
        pltpu.VMEM((4, JW), _i32),
        pltpu.VMEM((4, JW, 128), _i32),
        pltpu.VMEM((2, 2 * JW, IMG_H), _f32),
        pltpu.VMEM((2, P, JW), _f32),
        pltpu.SemaphoreType.DMA,
        pltpu.SemaphoreType.DMA,
        pltpu.SemaphoreType.DMA,
        pltpu.SemaphoreType.DMA,
    ],
)


GT_UNITS = NPATCH * 4
GT_T = (GT_UNITS + NW - 1) // NW
ZCHUNK_GT = 4096


def _gt_body(gtT_hbm, xpk_hbm, ypk_hbm, wtpk_hbm, accp_hbm,
             xv3, yv3, wt3, stg, valsS, zbuf, acc, sem, sem2):
    wid = _wid()
    sid = lax.axis_index("s")
    core = lax.axis_index("c")
    zf = jnp.zeros((16,), _f32)

    for t in range(GT_T):
        su = wid + NW * t

        @pl.when(su < GT_UNITS)
        def _():
            k = su // 4
            h = su % 4
            pltpu.sync_copy(xpk_hbm.at[k, h], xv3.at[t])
            pltpu.sync_copy(ypk_hbm.at[k, h], yv3.at[pl.ds(t * 64, 64)])
            pltpu.sync_copy(wtpk_hbm.at[k, h], wt3.at[t])

    def _zb(q, carry):
        zbuf[pl.ds(q * 16, 16)] = zf
        return carry
    lax.fori_loop(0, ZCHUNK_GT // 16, _zb, 0)

    def _batch(b, carry):
        for q in range(ACC_PER_TILE // ZCHUNK_GT):
            pltpu.sync_copy(zbuf, acc.at[pl.ds(sid * ACC_PER_TILE + q * ZCHUNK_GT, ZCHUNK_GT)])
        plsc.subcore_barrier()
        for t in range(GT_T):
            su = wid + NW * t

            @pl.when(su < GT_UNITS)
            def _():
                da = pltpu.async_copy(gtT_hbm.at[b].at[xv3.at[t, 0]], stg.at[0], sem)
                db = pltpu.async_copy(gtT_hbm.at[b].at[xv3.at[t, 1]], stg.at[1], sem)
                da.wait()
                db.wait()

                def _grp(g, carry2):
                    for q in range(8):
                        res = _blend_group(g, q, stg, yv3, xv3.at[t], t * 64)
                        w = plsc.bitcast(wt3[t, 32 + g, pl.ds(q * 16, 16)], _f32)
                        valsS[g, pl.ds(q * 16, 16)] = res * w
                    return carry2
                lax.fori_loop(0, 32, _grp, 0)

                descs = []
                for g in range(32):
                    descs.append(pltpu.async_copy(
                        valsS.at[g], acc.at[wt3.at[t, g]], sem2, add=True))
                for d in descs:
                    d.wait()
        plsc.subcore_barrier()
        for q in range(ACC_PER_TILE // ZCHUNK):
            off = sid * ACC_PER_TILE + q * ZCHUNK
            pltpu.sync_copy(acc.at[pl.ds(off, ZCHUNK)],
                            accp_hbm.at[core, b, pl.ds(off, ZCHUNK)])
        plsc.subcore_barrier()
        return carry

    lax.fori_loop(0, B, _batch, 0)


_sc_gt = pl.kernel(
    _gt_body,
    out_type=jax.ShapeDtypeStruct((NC, B, HW), _f32),
    mesh=_MESH,
    compiler_params=pltpu.CompilerParams(needs_layout_passes=False),
    scratch_types=[
        pltpu.VMEM((GT_T, 4, JW), _i32),
        pltpu.VMEM((GT_T * 64, 128), _i32),
        pltpu.VMEM((GT_T, 64, 128), _i32),
        pltpu.VMEM((2, JW, IMG_H), _f32),
        pltpu.VMEM((JW, 128), _f32),
        pltpu.VMEM((ZCHUNK_GT,), _f32),
        pltpu.VMEM_SHARED((HW,), _f32),
        pltpu.SemaphoreType.DMA,
        pltpu.SemaphoreType.DMA,
    ],
)


ROWCHUNK = 128


def _fin_body(accp_ref, cnt2_ref, out_ref):
    num = accp_ref[0, 0] + accp_ref[1, 0]
    den = jnp.maximum(cnt2_ref[0] + cnt2_ref[1], 1.0)
    out_ref[...] = (num / den)[None]


def _tc_fin(accp, cnt2p):
    nchunk = IMG_H // ROWCHUNK
    accp = accp.reshape(NC, B, IMG_H, IMG_W)
    cnt2p = cnt2p.reshape(NC, IMG_H, IMG_W)
    return pl.pallas_call(
        _fin_body,
        grid=(B, nchunk),
        in_specs=[
            pl.BlockSpec((NC, 1, ROWCHUNK, IMG_W), lambda b, j: (0, b, j, 0)),
            pl.BlockSpec((NC, ROWCHUNK, IMG_W), lambda b, j: (0, j, 0)),
        ],
        out_specs=pl.BlockSpec((1, ROWCHUNK, IMG_W), lambda b, j: (b, j, 0)),
        out_shape=jax.ShapeDtypeStruct((B, IMG_H, IMG_W), _f32),
    )(accp, cnt2p)


def _flatten_q(a):
    return (a.reshape(NPATCH, P, 4, JW).transpose(0, 2, 1, 3)
             .reshape(NPATCH, 4, JW, 128))


def _bits(a):
    return lax.bitcast_convert_type(a.astype(_f32), _i32)


def kernel(xb, gt, grid, x_idx, y_buf):
    g3 = grid.reshape(NPATCH, P, P, 2)
    gxk = g3[:, 0, :, 0]
    gyk = g3[..., 1]

    px = (gxk + 1.0) * 0.5 * (IMG_W - 1)
    x0 = jnp.floor(px)
    fx = px - x0
    x0i = x0.astype(_i32)
    x1i = x0i + 1
    wx0 = 1.0 - fx
    wx1 = jnp.where(x1i <= IMG_W - 1, fx, 0.0)
    x1c = jnp.clip(x1i, 0, IMG_W - 1)

    py = (gyk + 1.0) * 0.5 * (IMG_H - 1)
    y0 = jnp.floor(py)
    fy = (py - y0).astype(_f32)
    y0i = y0.astype(_i32)

    xpk = jnp.stack([x0i.reshape(NPATCH, 4, JW), x1c.reshape(NPATCH, 4, JW),
                     _bits(wx0.reshape(NPATCH, 4, JW)),
                     _bits(wx1.reshape(NPATCH, 4, JW))], axis=2)
    ypk = jnp.concatenate([_flatten_q(y0i), _bits(_flatten_q(fy))],
                          axis=2)
    y1c = jnp.minimum(y0i + 1, IMG_H - 1)
    wy0a = 1.0 - fy
    wy1a = jnp.where(y0i <= IMG_H - 2, fy, 0.0)
    ypk4 = jnp.stack([_flatten_q(y0i), _flatten_q(y1c),
                      _bits(_flatten_q(wy0a)), _bits(_flatten_q(wy1a))],
                     axis=2)

    xbT = xb.reshape(B * C, IMG_H, IMG_W).transpose(0, 2, 1)
    gtT = gt.transpose(0, 2, 1)

    w2304, tgt2304, cnt2p = _sc_pre(y_buf, x_idx)
    wtpk = jnp.concatenate([_flatten_q(tgt2304.reshape(NPATCH, P, P)),
                            _bits(_flatten_q(w2304.reshape(NPATCH, P, P)))],
                           axis=2)

    out1h = _sc_xb(xbT, xpk, ypk4)
    out1 = (out1h.reshape(B, C, NPATCH, 4, P, JW)
                 .transpose(0, 1, 2, 4, 3, 5)
                 .reshape(B, C, NPATCH, P, P))

    accp = _sc_gt(gtT, xpk, ypk, wtpk)
    out2 = _tc_fin(accp, cnt2p)
    return out1, out2
